# baseline (speedup 1.0000x reference)
import functools

import numpy as np
import jax
import jax.numpy as jnp
from jax import lax
from jax.experimental import pallas as pl
from jax.experimental.pallas import tpu as pltpu
from jax.experimental.pallas import tpu_sc as plsc

_B = 32
_C = 2304
_T = 64
_FOLD = _C // 64

_rng = np.random.RandomState(0)
_all = _rng.choice(_C, _FOLD * 2, replace=False)
_FWD_SET = set(np.sort(_all[:_FOLD]).tolist())
_BWD_SET = set(np.sort(_all[_FOLD:]).tolist())

_NW = 32
_ROW_W = 2 * _T
_ROWS_PER_B = _C // 2
_CHUNK_ROWS = 144
_N_CHUNKS = _ROWS_PER_B // _CHUNK_ROWS
_NBUF = 3

_PATCH = []
for _ch in range(_N_CHUNKS):
    _lst = []
    for _c in range(_ch * 2 * _CHUNK_ROWS, (_ch + 1) * 2 * _CHUNK_ROWS):
        if _c in _FWD_SET:
            _d = 1
        elif _c in _BWD_SET:
            _d = -1
        else:
            continue
        _lst.append(((_c // 2) % _CHUNK_ROWS, 64 * (_c % 2), _d))
    _PATCH.append(_lst)


def _patch_chunk(buf, scr, ch, lane):
    for (r, cb, d) in _PATCH[ch]:
        if d > 0:
            v0 = buf[r, pl.ds(cb + 1, 16)]
            v1 = buf[r, pl.ds(cb + 17, 16)]
            v2 = buf[r, pl.ds(cb + 33, 16)]
            if cb + 49 + 16 <= _ROW_W:
                v3 = buf[r, pl.ds(cb + 49, 16)]
            else:
                t = buf[r, pl.ds(cb + 48, 16)]
                scr[pl.ds(0, 16)] = t
                v3 = scr[pl.ds(1, 16)]
            v3 = jnp.where(lane == 15, 0.0, v3)
        else:
            if cb >= 1:
                v0 = buf[r, pl.ds(cb - 1, 16)]
            else:
                t = buf[r, pl.ds(cb, 16)]
                scr[pl.ds(1, 16)] = t
                v0 = scr[pl.ds(0, 16)]
            v0 = jnp.where(lane == 0, 0.0, v0)
            v1 = buf[r, pl.ds(cb + 15, 16)]
            v2 = buf[r, pl.ds(cb + 31, 16)]
            v3 = buf[r, pl.ds(cb + 47, 16)]
        buf[r, pl.ds(cb + 0, 16)] = v0
        buf[r, pl.ds(cb + 16, 16)] = v1
        buf[r, pl.ds(cb + 32, 16)] = v2
        buf[r, pl.ds(cb + 48, 16)] = v3


def _body(x_hbm, out_hbm, buf0, buf1, buf2, scr, si0, si1, si2, so0, so1, so2):
    lane = lax.iota(jnp.int32, 16)
    nc = 2
    wid = lax.axis_index("s") * nc + lax.axis_index("c")
    bufs = [buf0, buf1, buf2]
    sin = [si0, si1, si2]
    sout = [so0, so1, so2]

    base = wid * _ROWS_PER_B

    def load(ch):
        rb = base + ch * _CHUNK_ROWS
        b = ch % _NBUF
        return pltpu.async_copy(x_hbm.at[pl.ds(rb, _CHUNK_ROWS), :],
                                bufs[b], sin[b])

    def store(ch):
        rb = base + ch * _CHUNK_ROWS
        b = ch % _NBUF
        return pltpu.async_copy(bufs[b],
                                out_hbm.at[pl.ds(rb, _CHUNK_ROWS), :],
                                sout[b])

    in_h = [None] * _N_CHUNKS
    out_h = [None] * _N_CHUNKS
    for ch in range(min(2, _N_CHUNKS)):
        in_h[ch] = load(ch)
    for ch in range(_N_CHUNKS):
        in_h[ch].wait()
        _patch_chunk(bufs[ch % _NBUF], scr, ch, lane)
        out_h[ch] = store(ch)
        nxt = ch + 2
        if nxt < _N_CHUNKS:
            prev = nxt - _NBUF
            if prev >= 0:
                out_h[prev].wait()
            in_h[nxt] = load(nxt)
    for ch in range(_N_CHUNKS):
        if ch >= _N_CHUNKS - _NBUF:
            out_h[ch].wait()


_sc_call = functools.partial(
    pl.kernel,
    out_type=jax.ShapeDtypeStruct((_B * _ROWS_PER_B, _ROW_W), jnp.float32),
    mesh=plsc.VectorSubcoreMesh(core_axis_name="c", subcore_axis_name="s"),
    compiler_params=pltpu.CompilerParams(use_tc_tiling_on_sc=True,
                                         skip_device_barrier=True),
    scratch_types=[
        pltpu.VMEM((_CHUNK_ROWS, _ROW_W), jnp.float32),
        pltpu.VMEM((_CHUNK_ROWS, _ROW_W), jnp.float32),
        pltpu.VMEM((_CHUNK_ROWS, _ROW_W), jnp.float32),
        pltpu.VMEM((32,), jnp.float32),
        pltpu.SemaphoreType.DMA,
        pltpu.SemaphoreType.DMA,
        pltpu.SemaphoreType.DMA,
        pltpu.SemaphoreType.DMA,
        pltpu.SemaphoreType.DMA,
        pltpu.SemaphoreType.DMA,
    ],
)(_body)


@jax.jit
def kernel(x):
    rows = x.reshape(_B * _ROWS_PER_B, _ROW_W)
    out = _sc_call(rows)
    return out.reshape(x.shape)


# speedup vs baseline: 1.5227x; 1.5227x over previous
import functools

import numpy as np
import jax
import jax.numpy as jnp
from jax import lax
from jax.experimental import pallas as pl
from jax.experimental.pallas import tpu as pltpu
from jax.experimental.pallas import tpu_sc as plsc

_B = 32
_C = 2304
_T = 64
_FOLD = _C // 64

_rng = np.random.RandomState(0)
_all = _rng.choice(_C, _FOLD * 2, replace=False)
_FWD_SET = set(np.sort(_all[:_FOLD]).tolist())
_BWD_SET = set(np.sort(_all[_FOLD:]).tolist())

_NW = 32
_CHUNK_ROWS = 288
_N_CHUNKS = _C // _CHUNK_ROWS
_NBUF = 3

_PATCH = []
for _ch in range(_N_CHUNKS):
    _lst = []
    for _r in range(_CHUNK_ROWS):
        _c = _ch * _CHUNK_ROWS + _r
        if _c in _FWD_SET:
            _lst.append((_r, 1))
        elif _c in _BWD_SET:
            _lst.append((_r, -1))
    _PATCH.append(_lst)


def _patch_chunk(buf, scr, ch, lane):
    for (r, d) in _PATCH[ch]:
        if d > 0:
            v0 = buf[r, pl.ds(1, 16)]
            v1 = buf[r, pl.ds(17, 16)]
            v2 = buf[r, pl.ds(33, 16)]
            t = buf[r, pl.ds(48, 16)]
            scr[pl.ds(0, 16)] = t
            v3 = scr[pl.ds(1, 16)]
            v3 = jnp.where(lane == 15, 0.0, v3)
        else:
            t = buf[r, pl.ds(0, 16)]
            scr[pl.ds(1, 16)] = t
            v0 = scr[pl.ds(0, 16)]
            v0 = jnp.where(lane == 0, 0.0, v0)
            v1 = buf[r, pl.ds(15, 16)]
            v2 = buf[r, pl.ds(31, 16)]
            v3 = buf[r, pl.ds(47, 16)]
        buf[r, pl.ds(0, 16)] = v0
        buf[r, pl.ds(16, 16)] = v1
        buf[r, pl.ds(32, 16)] = v2
        buf[r, pl.ds(48, 16)] = v3


def _body(x_hbm, out_hbm, buf0, buf1, buf2, scr, si0, si1, si2, so0, so1, so2):
    lane = lax.iota(jnp.int32, 16)
    nc = 2
    wid = lax.axis_index("s") * nc + lax.axis_index("c")
    bufs = [buf0, buf1, buf2]
    sin = [si0, si1, si2]
    sout = [so0, so1, so2]

    base = wid * _C

    def load(ch):
        rb = base + ch * _CHUNK_ROWS
        b = ch % _NBUF
        return pltpu.async_copy(x_hbm.at[pl.ds(rb, _CHUNK_ROWS), :],
                                bufs[b], sin[b])

    def store(ch):
        rb = base + ch * _CHUNK_ROWS
        b = ch % _NBUF
        return pltpu.async_copy(bufs[b],
                                out_hbm.at[pl.ds(rb, _CHUNK_ROWS), :],
                                sout[b])

    in_h = [None] * _N_CHUNKS
    out_h = [None] * _N_CHUNKS
    for ch in range(min(2, _N_CHUNKS)):
        in_h[ch] = load(ch)
    for ch in range(_N_CHUNKS):
        in_h[ch].wait()
        _patch_chunk(bufs[ch % _NBUF], scr, ch, lane)
        out_h[ch] = store(ch)
        nxt = ch + 2
        if nxt < _N_CHUNKS:
            prev = nxt - _NBUF
            if prev >= 0:
                out_h[prev].wait()
            in_h[nxt] = load(nxt)
    for ch in range(_N_CHUNKS):
        if ch >= _N_CHUNKS - _NBUF:
            out_h[ch].wait()


_sc_call = functools.partial(
    pl.kernel,
    out_type=jax.ShapeDtypeStruct((_B * _C, _T), jnp.float32),
    mesh=plsc.VectorSubcoreMesh(core_axis_name="c", subcore_axis_name="s"),
    compiler_params=pltpu.CompilerParams(use_tc_tiling_on_sc=True,
                                         skip_device_barrier=True),
    scratch_types=[
        pltpu.VMEM((_CHUNK_ROWS, _T), jnp.float32),
        pltpu.VMEM((_CHUNK_ROWS, _T), jnp.float32),
        pltpu.VMEM((_CHUNK_ROWS, _T), jnp.float32),
        pltpu.VMEM((32,), jnp.float32),
        pltpu.SemaphoreType.DMA,
        pltpu.SemaphoreType.DMA,
        pltpu.SemaphoreType.DMA,
        pltpu.SemaphoreType.DMA,
        pltpu.SemaphoreType.DMA,
        pltpu.SemaphoreType.DMA,
    ],
)(_body)


@jax.jit
def kernel(x):
    rows = x.reshape(_B * _C, _T)
    out = _sc_call(rows)
    return out.reshape(x.shape)


# speedup vs baseline: 1.5308x; 1.0053x over previous
import functools

import numpy as np
import jax
import jax.numpy as jnp
from jax import lax
from jax.experimental import pallas as pl
from jax.experimental.pallas import tpu as pltpu
from jax.experimental.pallas import tpu_sc as plsc

_B = 32
_C = 2304
_T = 64
_FOLD = _C // 64

_rng = np.random.RandomState(0)
_all = _rng.choice(_C, _FOLD * 2, replace=False)
_FWD_SET = set(np.sort(_all[:_FOLD]).tolist())
_BWD_SET = set(np.sort(_all[_FOLD:]).tolist())

_NW = 32
_CHUNK_ROWS = 384
_N_CHUNKS = _C // _CHUNK_ROWS
_NBUF = 2

_PATCH = []
for _ch in range(_N_CHUNKS):
    _lst = []
    for _r in range(_CHUNK_ROWS):
        _c = _ch * _CHUNK_ROWS + _r
        if _c in _FWD_SET:
            _lst.append((_r, 1))
        elif _c in _BWD_SET:
            _lst.append((_r, -1))
    _PATCH.append(_lst)


def _patch_chunk(buf, scr, ch, lane):
    for (r, d) in _PATCH[ch]:
        if d > 0:
            v0 = buf[r, pl.ds(1, 16)]
            v1 = buf[r, pl.ds(17, 16)]
            v2 = buf[r, pl.ds(33, 16)]
            t = buf[r, pl.ds(48, 16)]
            scr[pl.ds(0, 16)] = t
            v3 = scr[pl.ds(1, 16)]
            v3 = jnp.where(lane == 15, 0.0, v3)
        else:
            t = buf[r, pl.ds(0, 16)]
            scr[pl.ds(1, 16)] = t
            v0 = scr[pl.ds(0, 16)]
            v0 = jnp.where(lane == 0, 0.0, v0)
            v1 = buf[r, pl.ds(15, 16)]
            v2 = buf[r, pl.ds(31, 16)]
            v3 = buf[r, pl.ds(47, 16)]
        buf[r, pl.ds(0, 16)] = v0
        buf[r, pl.ds(16, 16)] = v1
        buf[r, pl.ds(32, 16)] = v2
        buf[r, pl.ds(48, 16)] = v3


def _body(x_hbm, out_hbm, buf0, buf1, scr, si0, si1, so0, so1):
    lane = lax.iota(jnp.int32, 16)
    nc = 2
    wid = lax.axis_index("s") * nc + lax.axis_index("c")
    bufs = [buf0, buf1]
    sin = [si0, si1]
    sout = [so0, so1]

    base = wid * _C

    def load(ch):
        rb = base + ch * _CHUNK_ROWS
        b = ch % _NBUF
        return pltpu.async_copy(x_hbm.at[pl.ds(rb, _CHUNK_ROWS), :],
                                bufs[b], sin[b])

    def store(ch):
        rb = base + ch * _CHUNK_ROWS
        b = ch % _NBUF
        return pltpu.async_copy(bufs[b],
                                out_hbm.at[pl.ds(rb, _CHUNK_ROWS), :],
                                sout[b])

    in_h = [None] * _N_CHUNKS
    out_h = [None] * _N_CHUNKS
    for ch in range(min(2, _N_CHUNKS)):
        in_h[ch] = load(ch)
    for ch in range(_N_CHUNKS):
        in_h[ch].wait()
        _patch_chunk(bufs[ch % _NBUF], scr, ch, lane)
        out_h[ch] = store(ch)
        nxt = ch + 2
        if nxt < _N_CHUNKS:
            prev = nxt - _NBUF
            if prev >= 0:
                out_h[prev].wait()
            in_h[nxt] = load(nxt)
    for ch in range(_N_CHUNKS):
        if ch >= _N_CHUNKS - _NBUF:
            out_h[ch].wait()


_sc_call = functools.partial(
    pl.kernel,
    out_type=jax.ShapeDtypeStruct((_B * _C, _T), jnp.float32),
    mesh=plsc.VectorSubcoreMesh(core_axis_name="c", subcore_axis_name="s"),
    compiler_params=pltpu.CompilerParams(use_tc_tiling_on_sc=True,
                                         skip_device_barrier=True),
    scratch_types=[
        pltpu.VMEM((_CHUNK_ROWS, _T), jnp.float32),
        pltpu.VMEM((_CHUNK_ROWS, _T), jnp.float32),
        pltpu.VMEM((32,), jnp.float32),
        pltpu.SemaphoreType.DMA,
        pltpu.SemaphoreType.DMA,
        pltpu.SemaphoreType.DMA,
        pltpu.SemaphoreType.DMA,
    ],
)(_body)


@jax.jit
def kernel(x):
    rows = x.reshape(_B * _C, _T)
    out = _sc_call(rows)
    return out.reshape(x.shape)
